# baseline (speedup 1.0000x reference)
import functools

import jax
import jax.numpy as jnp
from jax import lax
from jax.experimental import pallas as pl
from jax.experimental.pallas import tpu as pltpu
from jax.experimental.pallas import tpu_sc as plsc

_BLK = 2000

_KCH = 48


def _pickv(v, j):
    out = v[15]
    for i in reversed(range(15)):
        out = jnp.where(j == i, v[i], out)
    return out


def _sc_segsum(z, cis_pad, s2s_pad, starts, ends, rounds, rw):
    n_out = rounds * 32 * rw
    cols = z.shape[1]
    acc_rows = rw + 1
    zeros = jnp.zeros((acc_rows * cols,), jnp.float32)
    elen = starts.shape[0]

    def body(z_hbm, cis_hbm, s2s_hbm, starts_hbm, ends_hbm, zeros_hbm,
             out_hbm, acc, cis_v, s2s_v, gidx2, lidx2, rows0, rows1,
             st_v, en_v, sem0, sem1):
        cid = lax.axis_index("c")
        sid = lax.axis_index("s")
        pltpu.sync_copy(starts_hbm, st_v)
        pltpu.sync_copy(ends_hbm, en_v)
        rbufs = (rows0, rows1)

        def round_fn(r, rcarry):
            t = r * 32 + cid * 16 + sid
            segbase = t * rw
            sa = st_v[pl.ds(r * 32 + cid * 16, 16)]
            sb = en_v[pl.ds(r * 32 + cid * 16, 16)]
            a = _pickv(sa, sid)
            b = _pickv(sb, sid)

            @pl.when(b > a)
            def _round():
                pltpu.sync_copy(zeros_hbm, acc)
                p0 = (a // 8) * 8
                npair = (b - p0 + 2 * _KCH - 1) // (2 * _KCH)

                def pair(pi, carry):
                    base = p0 + pi * 2 * _KCH
                    pltpu.sync_copy(cis_hbm.at[pl.ds(base, 2 * _KCH)], cis_v)
                    pltpu.sync_copy(s2s_hbm.at[pl.ds(base, 2 * _KCH)], s2s_v)
                    for h in range(2):
                        for j in range(_KCH // 16):
                            off = h * _KCH + j * 16
                            idx16 = cis_v[pl.ds(off, 16)]
                            s216 = s2s_v[pl.ds(off, 16)]
                            valid = (s216 >= segbase) & (s216 < segbase + rw)
                            gidx2[h, pl.ds(j * 16, 16)] = jnp.where(
                                valid, idx16, 0)
                            lidx2[h, pl.ds(j * 16, 16)] = jnp.where(
                                valid, s216 - segbase, rw)
                    cp0 = pltpu.async_copy(z_hbm.at[gidx2.at[0]], rows0, sem0)
                    cp1 = pltpu.async_copy(z_hbm.at[gidx2.at[1]], rows1, sem1)
                    for h, cp in ((0, cp0), (1, cp1)):
                        cp.wait()
                        rv = rbufs[h]
                        lgroups = [lidx2[h, pl.ds(j * 16, 16)]
                                   for j in range(_KCH // 16)]
                        for row in range(_KCH):
                            bse = lgroups[row // 16][row % 16] * cols
                            for c in range(cols // 16):
                                plsc.addupdate(
                                    acc.at[pl.ds(bse + c * 16, 16)],
                                    rv[row, pl.ds(c * 16, 16)])
                    return carry

                lax.fori_loop(0, npair, pair, 0)
                pltpu.sync_copy(acc.at[pl.ds(0, rw * cols)],
                                out_hbm.at[pl.ds(t * rw * cols, rw * cols)])

            return rcarry

        lax.fori_loop(0, rounds, round_fn, 0)

    out = pl.kernel(
        body,
        out_type=jax.ShapeDtypeStruct((n_out * cols,), jnp.float32),
        mesh=plsc.VectorSubcoreMesh(core_axis_name="c", subcore_axis_name="s"),
        compiler_params=pltpu.CompilerParams(use_tc_tiling_on_sc=False),
        scratch_types=[
            pltpu.VMEM((acc_rows * cols,), jnp.float32),
            pltpu.VMEM((2 * _KCH,), jnp.int32),
            pltpu.VMEM((2 * _KCH,), jnp.int32),
            pltpu.VMEM((2, _KCH), jnp.int32),
            pltpu.VMEM((2, _KCH), jnp.int32),
            pltpu.VMEM((_KCH, cols), jnp.float32),
            pltpu.VMEM((_KCH, cols), jnp.float32),
            pltpu.VMEM((elen,), jnp.int32),
            pltpu.VMEM((elen,), jnp.int32),
            pltpu.SemaphoreType.DMA,
            pltpu.SemaphoreType.DMA,
        ],
    )(z, cis_pad, s2s_pad, starts, ends, zeros)
    return out.reshape(n_out, cols)


def _segsum_edges(seg_sorted, rounds, rw):
    nr = rounds * 32
    qedges = jnp.arange(0, (nr + 1) * rw, rw, dtype=jnp.int32)
    e = jnp.searchsorted(seg_sorted, qedges).astype(jnp.int32)
    starts = jnp.zeros((nr,), jnp.int32).at[:nr].set(e[:nr])
    ends = jnp.zeros((nr,), jnp.int32).at[:nr].set(e[1:nr + 1])
    return starts, ends


def _leaky(x):
    return jnp.where(x >= 0, x, 0.1 * x)


def _p1_body(m_ref, f_ref, ds_ref, invc_ref, w1_ref, b1_ref, wo1t_ref,
             wp1_ref, bp1_ref, a1_ref, x1_ref, st_ref):
    step = pl.program_id(0)
    ident = _leaky(jnp.dot(f_ref[...], w1_ref[...],
                           preferred_element_type=jnp.float32) + b1_ref[0:1, :])
    a1_ref[...] = jnp.dot(ident, wo1t_ref[...], preferred_element_type=jnp.float32)
    x1 = _leaky(jnp.dot(ds_ref[...], wp1_ref[...],
                        preferred_element_type=jnp.float32) * invc_ref[...]
                + bp1_ref[0:1, :])
    x1_ref[...] = x1
    row = step * _BLK + lax.broadcasted_iota(jnp.int32, (_BLK, 1), 0)
    x1m = jnp.where(row < m_ref[0], x1, 0.0)

    @pl.when(step == 0)
    def _():
        st_ref[...] = jnp.zeros_like(st_ref)

    st_ref[0:1, :] += jnp.sum(x1m, axis=0, keepdims=True)
    st_ref[1:2, :] += jnp.sum(x1m * x1m, axis=0, keepdims=True)


def _p2_body(m_ref, x1_ref, a_ref, b_ref, wp2_ref, bp2_ref, x2_ref, st_ref):
    step = pl.program_id(0)
    y1 = x1_ref[...] * a_ref[0:1, :] + b_ref[0:1, :]
    x2 = _leaky(jnp.dot(y1, wp2_ref[...],
                        preferred_element_type=jnp.float32) + bp2_ref[0:1, :])
    x2_ref[...] = x2
    row = step * _BLK + lax.broadcasted_iota(jnp.int32, (_BLK, 1), 0)
    x2m = jnp.where(row < m_ref[0], x2, 0.0)

    @pl.when(step == 0)
    def _():
        st_ref[...] = jnp.zeros_like(st_ref)

    st_ref[0:1, :] += jnp.sum(x2m, axis=0, keepdims=True)
    st_ref[1:2, :] += jnp.sum(x2m * x2m, axis=0, keepdims=True)


def _p3_body(x2_ref, a_ref, b_ref, wp3_ref, bp3_ref, h_ref):
    y2 = x2_ref[...] * a_ref[0:1, :] + b_ref[0:1, :]
    h_ref[...] = _leaky(jnp.dot(y2, wp3_ref[...],
                                preferred_element_type=jnp.float32) + bp3_ref[0:1, :])


def _p4_body(a1_ref, hg_ref, wo1b_ref, bo1_ref, wo2_ref, bo2_ref, z_ref):
    u = a1_ref[...] + jnp.dot(hg_ref[...], wo1b_ref[...],
                              preferred_element_type=jnp.float32)
    z1 = _leaky(u + bo1_ref[0:1, :])
    z_ref[...] = jnp.dot(z1, wo2_ref[...],
                         preferred_element_type=jnp.float32) + bo2_ref[0:1, :]


def _row8(v):
    return jnp.broadcast_to(v[None, :], (8, v.shape[0]))


def _full(shape):
    return pl.BlockSpec(shape, lambda i: (0, 0))


def _rows(c):
    return pl.BlockSpec((_BLK, c), lambda i: (i, 0))


def kernel(features, coors, coors_inv, scale_2_coors_inv,
           W1, b1, Wp1, bp1, g1, beta1, Wp2, bp2, g2, beta2, Wp3, bp3,
           Wo1, bo1, Wo2, bo2):
    n = features.shape[0]
    in_c = features.shape[1]
    out_c = W1.shape[1]
    hid = Wp1.shape[1]
    m_out = 25000
    grid = n // _BLK

    c32 = coors.astype(jnp.int32)
    key = ((c32[:, 0] << 27) | ((c32[:, 1] >> 1) << 18)
           | ((c32[:, 2] >> 1) << 9) | (c32[:, 3] >> 1))
    sk, perm = lax.sort_key_val(key, jnp.arange(n, dtype=jnp.int32))
    newseg = jnp.concatenate(
        [jnp.ones((1,), jnp.int32), (sk[1:] != sk[:-1]).astype(jnp.int32)])
    ranks = jnp.cumsum(newseg) - 1
    inv = jnp.take(ranks, jnp.searchsorted(sk, key))
    n_valid = (ranks[-1] + 1).astype(jnp.float32)
    m_arr = ranks[-1:] + 1

    pad = 128
    perm_pad = jnp.concatenate([perm, jnp.zeros((pad,), jnp.int32)])
    ranks_pad = jnp.concatenate(
        [ranks, jnp.full((pad,), jnp.int32(0x3FFFFFFF))])
    st_a, en_a = _segsum_edges(ranks, 9, 368)
    ds = _sc_segsum(features, perm_pad, ranks_pad, st_a, en_a, 9, 368)[:n]
    edges_n = jnp.searchsorted(
        ranks, jnp.arange(n + 1, dtype=jnp.int32)).astype(jnp.int32)
    cnt = (edges_n[1:] - edges_n[:-1]).astype(jnp.float32)
    invc = 1.0 / jnp.maximum(cnt, 1.0)
    invc_h = jnp.broadcast_to(invc[:, None], (n, Wp1.shape[1]))

    wo1t = Wo1[:out_c]
    wo1b = Wo1[out_c:]

    a1, x1, st1 = pl.pallas_call(
        _p1_body,
        grid=(grid,),
        in_specs=[
            pl.BlockSpec(memory_space=pltpu.SMEM),
            _rows(in_c), _rows(in_c), _rows(hid),
            _full((in_c, out_c)), _full((8, out_c)),
            _full((out_c, out_c)), _full((in_c, hid)), _full((8, hid)),
        ],
        out_specs=[_rows(out_c), _rows(hid),
                   pl.BlockSpec((8, hid), lambda i: (0, 0))],
        out_shape=[
            jax.ShapeDtypeStruct((n, out_c), jnp.float32),
            jax.ShapeDtypeStruct((n, hid), jnp.float32),
            jax.ShapeDtypeStruct((8, hid), jnp.float32),
        ],
        compiler_params=pltpu.CompilerParams(
            dimension_semantics=("arbitrary",)),
    )(m_arr, features, ds, invc_h, W1, _row8(b1), wo1t, Wp1, _row8(bp1))

    mean1 = st1[0] / n_valid
    var1 = jnp.maximum(st1[1] / n_valid - mean1 * mean1, 0.0)
    sc1 = g1 / jnp.sqrt(var1 + 1e-5)
    sh1 = beta1 - mean1 * sc1

    x2, st2 = pl.pallas_call(
        _p2_body,
        grid=(grid,),
        in_specs=[
            pl.BlockSpec(memory_space=pltpu.SMEM),
            _rows(hid), _full((8, hid)), _full((8, hid)),
            _full((hid, hid)), _full((8, hid)),
        ],
        out_specs=[_rows(hid), pl.BlockSpec((8, hid), lambda i: (0, 0))],
        out_shape=[
            jax.ShapeDtypeStruct((n, hid), jnp.float32),
            jax.ShapeDtypeStruct((8, hid), jnp.float32),
        ],
        compiler_params=pltpu.CompilerParams(
            dimension_semantics=("arbitrary",)),
    )(m_arr, x1, _row8(sc1), _row8(sh1), Wp2, _row8(bp2))

    mean2 = st2[0] / n_valid
    var2 = jnp.maximum(st2[1] / n_valid - mean2 * mean2, 0.0)
    sc2 = g2 / jnp.sqrt(var2 + 1e-5)
    sh2 = beta2 - mean2 * sc2

    h = pl.pallas_call(
        _p3_body,
        grid=(grid,),
        in_specs=[_rows(hid), _full((8, hid)), _full((8, hid)),
                  _full((hid, out_c)), _full((8, out_c))],
        out_specs=_rows(out_c),
        out_shape=jax.ShapeDtypeStruct((n, out_c), jnp.float32),
        compiler_params=pltpu.CompilerParams(
            dimension_semantics=("arbitrary",)),
    )(x2, _row8(sc2), _row8(sh2), Wp3, _row8(bp3))

    hg = h[inv]

    z = pl.pallas_call(
        _p4_body,
        grid=(grid,),
        in_specs=[_rows(out_c), _rows(out_c), _full((out_c, out_c)),
                  _full((8, out_c)), _full((out_c, out_c)), _full((8, out_c))],
        out_specs=_rows(out_c),
        out_shape=jax.ShapeDtypeStruct((n, out_c), jnp.float32),
        compiler_params=pltpu.CompilerParams(
            dimension_semantics=("arbitrary",)),
    )(a1, hg, wo1b, _row8(bo1), Wo2, _row8(bo2))

    zs = jax.ops.segment_sum(z[coors_inv], scale_2_coors_inv,
                             num_segments=m_out)
    cnt2 = jax.ops.segment_sum(jnp.ones((coors_inv.shape[0],), jnp.float32),
                               scale_2_coors_inv, num_segments=m_out)
    return zs / jnp.maximum(cnt2, 1.0)[:, None]


# speedup vs baseline: 4.4055x; 4.4055x over previous
import functools

import jax
import jax.numpy as jnp
from jax import lax
from jax.experimental import pallas as pl
from jax.experimental.pallas import tpu as pltpu
from jax.experimental.pallas import tpu_sc as plsc

_BLK = 2000

_KCH = 48


def _pickv(v, j):
    out = v[15]
    for i in reversed(range(15)):
        out = jnp.where(j == i, v[i], out)
    return out


def _sc_segsum(z, cis_pad, s2s_pad, starts, ends, rounds, rw):
    n_out = rounds * 32 * rw
    cols = z.shape[1]
    acc_rows = rw + 1
    zeros = jnp.zeros((acc_rows * cols,), jnp.float32)
    elen = starts.shape[0]

    def body(z_hbm, cis_hbm, s2s_hbm, starts_hbm, ends_hbm, zeros_hbm,
             out_hbm, acc, cntacc, cis_v, s2s_v, gidx2, lidx2, rows0, rows1,
             st_v, en_v, sem0, sem1):
        cid = lax.axis_index("c")
        sid = lax.axis_index("s")
        pltpu.sync_copy(starts_hbm, st_v)
        pltpu.sync_copy(ends_hbm, en_v)
        rbufs = (rows0, rows1)
        lane = lax.broadcasted_iota(jnp.int32, (16,), 0)
        one0 = jnp.where(lane == 0, 1.0, 0.0).astype(jnp.float32)

        def round_fn(r, rcarry):
            t = r * 32 + cid * 16 + sid
            segbase = t * rw
            sa = st_v[pl.ds(r * 32 + cid * 16, 16)]
            sb = en_v[pl.ds(r * 32 + cid * 16, 16)]
            a = _pickv(sa, sid)
            b = _pickv(sb, sid)

            @pl.when(b > a)
            def _round():
                pltpu.sync_copy(zeros_hbm, acc)
                pltpu.sync_copy(zeros_hbm.at[pl.ds(0, acc_rows * 16)], cntacc)
                p0 = (a // 8) * 8
                npair = (b - p0 + 2 * _KCH - 1) // (2 * _KCH)

                def pair(pi, carry):
                    base = p0 + pi * 2 * _KCH
                    pltpu.sync_copy(cis_hbm.at[pl.ds(base, 2 * _KCH)], cis_v)
                    pltpu.sync_copy(s2s_hbm.at[pl.ds(base, 2 * _KCH)], s2s_v)
                    for h in range(2):
                        for j in range(_KCH // 16):
                            off = h * _KCH + j * 16
                            idx16 = cis_v[pl.ds(off, 16)]
                            s216 = s2s_v[pl.ds(off, 16)]
                            valid = (s216 >= segbase) & (s216 < segbase + rw)
                            gidx2[h, pl.ds(j * 16, 16)] = jnp.where(
                                valid, idx16, 0)
                            lidx2[h, pl.ds(j * 16, 16)] = jnp.where(
                                valid, s216 - segbase, rw)
                    cp0 = pltpu.async_copy(z_hbm.at[gidx2.at[0]], rows0, sem0)
                    cp1 = pltpu.async_copy(z_hbm.at[gidx2.at[1]], rows1, sem1)
                    for h, cp in ((0, cp0), (1, cp1)):
                        cp.wait()
                        rv = rbufs[h]
                        lgroups = [lidx2[h, pl.ds(j * 16, 16)]
                                   for j in range(_KCH // 16)]
                        for row in range(_KCH):
                            lid = lgroups[row // 16][row % 16]
                            bse = lid * cols
                            plsc.addupdate(
                                cntacc.at[pl.ds(lid * 16, 16)], one0)
                            for c in range(cols // 16):
                                plsc.addupdate(
                                    acc.at[pl.ds(bse + c * 16, 16)],
                                    rv[row, pl.ds(c * 16, 16)])
                    return carry

                lax.fori_loop(0, npair, pair, 0)

                def divrow(rr, c2):
                    cv = cntacc[pl.ds(rr * 16, 16)]
                    rcpv = jnp.full((16,), 1.0, jnp.float32) / jnp.maximum(cv, 1.0)
                    rcp = rcpv[0]
                    for c in range(cols // 16):
                        off = rr * cols + c * 16
                        acc[pl.ds(off, 16)] = acc[pl.ds(off, 16)] * rcp
                    return c2

                lax.fori_loop(0, rw, divrow, 0)
                pltpu.sync_copy(acc.at[pl.ds(0, rw * cols)],
                                out_hbm.at[pl.ds(t * rw * cols, rw * cols)])

            return rcarry

        lax.fori_loop(0, rounds, round_fn, 0)

    out = pl.kernel(
        body,
        out_type=jax.ShapeDtypeStruct((n_out * cols,), jnp.float32),
        mesh=plsc.VectorSubcoreMesh(core_axis_name="c", subcore_axis_name="s"),
        compiler_params=pltpu.CompilerParams(use_tc_tiling_on_sc=False),
        scratch_types=[
            pltpu.VMEM((acc_rows * cols,), jnp.float32),
            pltpu.VMEM((acc_rows * 16,), jnp.float32),
            pltpu.VMEM((2 * _KCH,), jnp.int32),
            pltpu.VMEM((2 * _KCH,), jnp.int32),
            pltpu.VMEM((2, _KCH), jnp.int32),
            pltpu.VMEM((2, _KCH), jnp.int32),
            pltpu.VMEM((_KCH, cols), jnp.float32),
            pltpu.VMEM((_KCH, cols), jnp.float32),
            pltpu.VMEM((elen,), jnp.int32),
            pltpu.VMEM((elen,), jnp.int32),
            pltpu.SemaphoreType.DMA,
            pltpu.SemaphoreType.DMA,
        ],
    )(z, cis_pad, s2s_pad, starts, ends, zeros)
    return out.reshape(n_out, cols)


def _segsum_edges(seg_sorted, rounds, rw):
    nr = rounds * 32
    qedges = jnp.arange(0, (nr + 1) * rw, rw, dtype=jnp.int32)
    e = jnp.searchsorted(seg_sorted, qedges).astype(jnp.int32)
    starts = jnp.zeros((nr,), jnp.int32).at[:nr].set(e[:nr])
    ends = jnp.zeros((nr,), jnp.int32).at[:nr].set(e[1:nr + 1])
    return starts, ends


def _leaky(x):
    return jnp.where(x >= 0, x, 0.1 * x)


def _p1_body(m_ref, f_ref, ds_ref, w1_ref, b1_ref, wo1t_ref,
             wp1_ref, bp1_ref, a1_ref, x1_ref, st_ref):
    step = pl.program_id(0)
    ident = _leaky(jnp.dot(f_ref[...], w1_ref[...],
                           preferred_element_type=jnp.float32) + b1_ref[0:1, :])
    a1_ref[...] = jnp.dot(ident, wo1t_ref[...], preferred_element_type=jnp.float32)
    x1 = _leaky(jnp.dot(ds_ref[...], wp1_ref[...],
                        preferred_element_type=jnp.float32) + bp1_ref[0:1, :])
    x1_ref[...] = x1
    row = step * _BLK + lax.broadcasted_iota(jnp.int32, (_BLK, 1), 0)
    x1m = jnp.where(row < m_ref[0], x1, 0.0)

    @pl.when(step == 0)
    def _():
        st_ref[...] = jnp.zeros_like(st_ref)

    st_ref[0:1, :] += jnp.sum(x1m, axis=0, keepdims=True)
    st_ref[1:2, :] += jnp.sum(x1m * x1m, axis=0, keepdims=True)


def _p2_body(m_ref, x1_ref, a_ref, b_ref, wp2_ref, bp2_ref, x2_ref, st_ref):
    step = pl.program_id(0)
    y1 = x1_ref[...] * a_ref[0:1, :] + b_ref[0:1, :]
    x2 = _leaky(jnp.dot(y1, wp2_ref[...],
                        preferred_element_type=jnp.float32) + bp2_ref[0:1, :])
    x2_ref[...] = x2
    row = step * _BLK + lax.broadcasted_iota(jnp.int32, (_BLK, 1), 0)
    x2m = jnp.where(row < m_ref[0], x2, 0.0)

    @pl.when(step == 0)
    def _():
        st_ref[...] = jnp.zeros_like(st_ref)

    st_ref[0:1, :] += jnp.sum(x2m, axis=0, keepdims=True)
    st_ref[1:2, :] += jnp.sum(x2m * x2m, axis=0, keepdims=True)


def _p3_body(x2_ref, a_ref, b_ref, wp3_ref, bp3_ref, h_ref):
    y2 = x2_ref[...] * a_ref[0:1, :] + b_ref[0:1, :]
    h_ref[...] = _leaky(jnp.dot(y2, wp3_ref[...],
                                preferred_element_type=jnp.float32) + bp3_ref[0:1, :])


def _p4_body(a1_ref, hg_ref, wo1b_ref, bo1_ref, wo2_ref, bo2_ref, z_ref):
    u = a1_ref[...] + jnp.dot(hg_ref[...], wo1b_ref[...],
                              preferred_element_type=jnp.float32)
    z1 = _leaky(u + bo1_ref[0:1, :])
    z_ref[...] = jnp.dot(z1, wo2_ref[...],
                         preferred_element_type=jnp.float32) + bo2_ref[0:1, :]


def _row8(v):
    return jnp.broadcast_to(v[None, :], (8, v.shape[0]))


def _full(shape):
    return pl.BlockSpec(shape, lambda i: (0, 0))


def _rows(c):
    return pl.BlockSpec((_BLK, c), lambda i: (i, 0))


def kernel(features, coors, coors_inv, scale_2_coors_inv,
           W1, b1, Wp1, bp1, g1, beta1, Wp2, bp2, g2, beta2, Wp3, bp3,
           Wo1, bo1, Wo2, bo2):
    n = features.shape[0]
    in_c = features.shape[1]
    out_c = W1.shape[1]
    hid = Wp1.shape[1]
    m_out = 25000
    grid = n // _BLK

    c32 = coors.astype(jnp.int32)
    key = ((c32[:, 0] << 27) | ((c32[:, 1] >> 1) << 18)
           | ((c32[:, 2] >> 1) << 9) | (c32[:, 3] >> 1))
    sk, perm = lax.sort_key_val(key, jnp.arange(n, dtype=jnp.int32))
    newseg = jnp.concatenate(
        [jnp.ones((1,), jnp.int32), (sk[1:] != sk[:-1]).astype(jnp.int32)])
    ranks = jnp.cumsum(newseg) - 1
    inv = jnp.zeros((n,), jnp.int32).at[perm].set(ranks)
    n_valid = (ranks[-1] + 1).astype(jnp.float32)
    m_arr = ranks[-1:] + 1

    pad = 128
    perm_pad = jnp.concatenate([perm, jnp.zeros((pad,), jnp.int32)])
    ranks_pad = jnp.concatenate(
        [ranks, jnp.full((pad,), jnp.int32(0x3FFFFFFF))])
    st_a, en_a = _segsum_edges(ranks, 9, 368)
    ds = _sc_segsum(features, perm_pad, ranks_pad, st_a, en_a, 9, 368)[:n]

    wo1t = Wo1[:out_c]
    wo1b = Wo1[out_c:]

    a1, x1, st1 = pl.pallas_call(
        _p1_body,
        grid=(grid,),
        in_specs=[
            pl.BlockSpec(memory_space=pltpu.SMEM),
            _rows(in_c), _rows(in_c),
            _full((in_c, out_c)), _full((8, out_c)),
            _full((out_c, out_c)), _full((in_c, hid)), _full((8, hid)),
        ],
        out_specs=[_rows(out_c), _rows(hid),
                   pl.BlockSpec((8, hid), lambda i: (0, 0))],
        out_shape=[
            jax.ShapeDtypeStruct((n, out_c), jnp.float32),
            jax.ShapeDtypeStruct((n, hid), jnp.float32),
            jax.ShapeDtypeStruct((8, hid), jnp.float32),
        ],
        compiler_params=pltpu.CompilerParams(
            dimension_semantics=("arbitrary",)),
    )(m_arr, features, ds, W1, _row8(b1), wo1t, Wp1, _row8(bp1))

    mean1 = st1[0] / n_valid
    var1 = jnp.maximum(st1[1] / n_valid - mean1 * mean1, 0.0)
    sc1 = g1 / jnp.sqrt(var1 + 1e-5)
    sh1 = beta1 - mean1 * sc1

    x2, st2 = pl.pallas_call(
        _p2_body,
        grid=(grid,),
        in_specs=[
            pl.BlockSpec(memory_space=pltpu.SMEM),
            _rows(hid), _full((8, hid)), _full((8, hid)),
            _full((hid, hid)), _full((8, hid)),
        ],
        out_specs=[_rows(hid), pl.BlockSpec((8, hid), lambda i: (0, 0))],
        out_shape=[
            jax.ShapeDtypeStruct((n, hid), jnp.float32),
            jax.ShapeDtypeStruct((8, hid), jnp.float32),
        ],
        compiler_params=pltpu.CompilerParams(
            dimension_semantics=("arbitrary",)),
    )(m_arr, x1, _row8(sc1), _row8(sh1), Wp2, _row8(bp2))

    mean2 = st2[0] / n_valid
    var2 = jnp.maximum(st2[1] / n_valid - mean2 * mean2, 0.0)
    sc2 = g2 / jnp.sqrt(var2 + 1e-5)
    sh2 = beta2 - mean2 * sc2

    h = pl.pallas_call(
        _p3_body,
        grid=(grid,),
        in_specs=[_rows(hid), _full((8, hid)), _full((8, hid)),
                  _full((hid, out_c)), _full((8, out_c))],
        out_specs=_rows(out_c),
        out_shape=jax.ShapeDtypeStruct((n, out_c), jnp.float32),
        compiler_params=pltpu.CompilerParams(
            dimension_semantics=("arbitrary",)),
    )(x2, _row8(sc2), _row8(sh2), Wp3, _row8(bp3))

    hg = h[inv]

    z = pl.pallas_call(
        _p4_body,
        grid=(grid,),
        in_specs=[_rows(out_c), _rows(out_c), _full((out_c, out_c)),
                  _full((8, out_c)), _full((out_c, out_c)), _full((8, out_c))],
        out_specs=_rows(out_c),
        out_shape=jax.ShapeDtypeStruct((n, out_c), jnp.float32),
        compiler_params=pltpu.CompilerParams(
            dimension_semantics=("arbitrary",)),
    )(a1, hg, wo1b, _row8(bo1), Wo2, _row8(bo2))

    zs = jax.ops.segment_sum(z[coors_inv], scale_2_coors_inv,
                             num_segments=m_out)
    cnt2 = jax.ops.segment_sum(jnp.ones((coors_inv.shape[0],), jnp.float32),
                               scale_2_coors_inv, num_segments=m_out)
    return zs / jnp.maximum(cnt2, 1.0)[:, None]
